# Initial kernel scaffold; baseline (speedup 1.0000x reference)
import jax
import jax.numpy as jnp
from jax.experimental import pallas as pl


def kernel(x, edge_index, edge_attr, ea_W1, ea_b1, ea_W2, ea_b2, tag_W, tag_b, out_W, out_b):
    raise NotImplementedError("write your pallas kernel here")


# speedup vs baseline: 1.4008x; 1.4008x over previous
import jax
import jax.numpy as jnp
from jax.experimental import pallas as pl

N = 10000
E = 640000
H = 128
EDGE_TILE = 2560


def _mlp_body(xc_ref, xr_ref, ea_ref, wc_ref, wr_ref, we_ref, b1_ref, out_ref):
    acc = jnp.dot(xc_ref[...], wc_ref[...], preferred_element_type=jnp.float32)
    acc = acc + jnp.dot(xr_ref[...], wr_ref[...], preferred_element_type=jnp.float32)
    acc = acc + jnp.dot(ea_ref[...], we_ref[...], preferred_element_type=jnp.float32)
    out_ref[...] = jnp.maximum(acc + b1_ref[...], 0.0)


def _edge_mlp(xc, xr, ea, wc, wr, we, b1):
    grid = (E // EDGE_TILE,)
    return pl.pallas_call(
        _mlp_body,
        grid=grid,
        in_specs=[
            pl.BlockSpec((EDGE_TILE, 8), lambda i: (i, 0)),
            pl.BlockSpec((EDGE_TILE, 8), lambda i: (i, 0)),
            pl.BlockSpec((EDGE_TILE, 4), lambda i: (i, 0)),
            pl.BlockSpec((8, H), lambda i: (0, 0)),
            pl.BlockSpec((8, H), lambda i: (0, 0)),
            pl.BlockSpec((4, H), lambda i: (0, 0)),
            pl.BlockSpec((1, H), lambda i: (0, 0)),
        ],
        out_specs=pl.BlockSpec((EDGE_TILE, H), lambda i: (i, 0)),
        out_shape=jax.ShapeDtypeStruct((E, H), jnp.float32),
    )(xc, xr, ea, wc, wr, we, b1)


def kernel(x, edge_index, edge_attr, ea_W1, ea_b1, ea_W2, ea_b2, tag_W, tag_b, out_W, out_b):
    row = edge_index[0]
    col = edge_index[1]

    deg = jnp.zeros((N,), jnp.float32).at[col].add(1.0)
    dinv = jnp.where(deg > 0, jax.lax.rsqrt(jnp.where(deg > 0, deg, 1.0)), 0.0)

    x8 = jnp.pad(x, ((0, 0), (0, 1)))
    xc = x8[col]
    xr = x8[row]

    wc = jnp.pad(ea_W1[0:7], ((0, 1), (0, 0)))
    wr = jnp.pad(ea_W1[7:14], ((0, 1), (0, 0)))
    we = ea_W1[14:18]

    h1 = _edge_mlp(xc, xr, ea_attr := edge_attr, wc, wr, we, ea_b1.reshape(1, H))

    S = jnp.zeros((N, H), jnp.float32).at[col].add(h1)
    h = S @ ea_W2 + deg[:, None] * ea_b2[None, :]

    L = tag_W.shape[0]
    for l in range(L):
        out = h @ tag_W[l, 0]
        xk = h
        for k in range(1, tag_W.shape[1]):
            z = dinv[:, None] * xk
            agg = jnp.zeros((N, H), jnp.float32).at[col].add(z[row])
            xk = dinv[:, None] * agg
            out = out + xk @ tag_W[l, k]
        h = out + tag_b[l]
        if l < L - 1:
            h = jnp.maximum(h, 0.0)
    y = h @ out_W + out_b
    return y.reshape(1, -1)


# speedup vs baseline: 12.4935x; 8.9187x over previous
import functools

import jax
import jax.numpy as jnp
from jax import lax
from jax.experimental import pallas as pl
from jax.experimental.pallas import tpu as pltpu
from jax.experimental.pallas import tpu_sc as plsc

N = 10000
E = 640000
H = 128
NPAD = 112
NA = N + NPAD
NC = 2
NS = 16
NW = NC * NS
CW = 128
EPW = 157 * CW
E_PAD = NW * EPW
NA_PER_S = NA // NS

MLP_TILE = 2048
TN = 1000

_mesh = plsc.VectorSubcoreMesh(core_axis_name="c", subcore_axis_name="s")


def _worker_id():
    return lax.axis_index("s") * NC + lax.axis_index("c")


@functools.partial(
    pl.kernel,
    mesh=_mesh,
    out_type=[
        jax.ShapeDtypeStruct((8, E_PAD), jnp.float32),
        jax.ShapeDtypeStruct((8, E_PAD), jnp.float32),
        jax.ShapeDtypeStruct((NW, 1, NA), jnp.float32),
    ],
    scratch_types=[
        pltpu.VMEM((NA * 8,), jnp.float32),
        pltpu.VMEM((CW,), jnp.int32),
        pltpu.VMEM((CW,), jnp.int32),
        pltpu.VMEM((8, CW), jnp.float32),
        pltpu.VMEM((8, CW), jnp.float32),
        pltpu.VMEM((NA,), jnp.float32),
    ],
    compiler_params=pltpu.CompilerParams(needs_layout_passes=False),
)
def _sc_gather_x(row_hbm, col_hbm, x8_hbm,
                 xc_hbm, xr_hbm, deg_hbm,
                 x8v, cidx, ridx, cbuf, rbuf, dacc):
    wid = _worker_id()
    base = wid * EPW

    def zbody(i, _):
        dacc[pl.ds(i * 16, 16)] = jnp.zeros((16,), jnp.float32)
        return _

    lax.fori_loop(0, NA // 16, zbody, None)
    pltpu.sync_copy(x8_hbm, x8v)
    ones16 = jnp.ones((16,), jnp.float32)

    def body(i, _):
        off = base + i * CW
        pltpu.sync_copy(col_hbm.at[pl.ds(off, CW)], cidx)
        pltpu.sync_copy(row_hbm.at[pl.ds(off, CW)], ridx)
        for j in range(CW // 16):
            c16 = cidx[pl.ds(j * 16, 16)]
            r16 = ridx[pl.ds(j * 16, 16)]
            plsc.addupdate_scatter(dacc, [c16], ones16)
            c8 = c16 * 8
            r8 = r16 * 8
            for f in range(8):
                cbuf[f, pl.ds(j * 16, 16)] = plsc.load_gather(x8v, [c8 + f])
                rbuf[f, pl.ds(j * 16, 16)] = plsc.load_gather(x8v, [r8 + f])
        pltpu.sync_copy(cbuf, xc_hbm.at[:, pl.ds(off, CW)])
        pltpu.sync_copy(rbuf, xr_hbm.at[:, pl.ds(off, CW)])
        return _

    lax.fori_loop(0, EPW // CW, body, None)
    pltpu.sync_copy(dacc, deg_hbm.at[wid, 0])


def _make_sc_aggregate(gather_src: bool):
    scratch = [
        pltpu.VMEM((CW,), jnp.int32),
        pltpu.VMEM((CW,), jnp.int32),
        pltpu.VMEM((CW,), jnp.int32),
        pltpu.VMEM((CW,), jnp.int32),
        pltpu.VMEM((CW, H), jnp.float32),
        pltpu.VMEM((CW, H), jnp.float32),
        pltpu.VMEM_SHARED((NA, H), jnp.float32),
        pltpu.SemaphoreType.DMA,
        pltpu.SemaphoreType.DMA,
    ]

    @functools.partial(
        pl.kernel,
        mesh=_mesh,
        out_type=jax.ShapeDtypeStruct((NC, NA, H), jnp.float32),
        scratch_types=scratch,
    )
    def agg(src_hbm, row_hbm, col_hbm, zeros_hbm, out_hbm,
            cidx0, cidx1, ridx0, ridx1, buf0, buf1, acc, sem0, sem1):
        cid = lax.axis_index("c")
        sid = lax.axis_index("s")
        base = _worker_id() * EPW

        pltpu.sync_copy(zeros_hbm.at[pl.ds(sid * NA_PER_S, NA_PER_S)],
                        acc.at[pl.ds(sid * NA_PER_S, NA_PER_S)])
        plsc.subcore_barrier()

        def start(i, cidx, ridx, buf, sem):
            off = base + i * CW
            pltpu.sync_copy(col_hbm.at[pl.ds(off, CW)], cidx)
            if gather_src:
                pltpu.sync_copy(row_hbm.at[pl.ds(off, CW)], ridx)
                pltpu.async_copy(src_hbm.at[ridx], buf, sem)
            else:
                pltpu.async_copy(src_hbm.at[pl.ds(off, CW)], buf, sem)

        def finish(cidx, ridx, buf, sem):
            if gather_src:
                pltpu.make_async_copy(src_hbm.at[ridx], buf, sem).wait()
            else:
                pltpu.make_async_copy(src_hbm.at[pl.ds(0, CW)], buf, sem).wait()
            pltpu.sync_copy(buf, acc.at[cidx], add=True)

        start(0, cidx0, ridx0, buf0, sem0)

        def body(k, _):
            i0 = 2 * k
            start(i0 + 1, cidx1, ridx1, buf1, sem1)
            finish(cidx0, ridx0, buf0, sem0)

            @pl.when(i0 + 2 < EPW // CW)
            def _():
                start(i0 + 2, cidx0, ridx0, buf0, sem0)

            finish(cidx1, ridx1, buf1, sem1)
            return _

        lax.fori_loop(0, (EPW // CW) // 2, body, None)
        finish(cidx0, ridx0, buf0, sem0)

        plsc.subcore_barrier()
        pltpu.sync_copy(acc.at[pl.ds(sid * NA_PER_S, NA_PER_S)],
                        out_hbm.at[cid, pl.ds(sid * NA_PER_S, NA_PER_S)])

    return agg


_sc_agg_linear = _make_sc_aggregate(gather_src=False)
_sc_agg_gather = _make_sc_aggregate(gather_src=True)


def _mlp_body(xc_ref, xr_ref, ea_ref, wc_ref, wr_ref, we_ref, b1_ref, out_ref):
    dn = (((0,), (0,)), ((), ()))
    acc = lax.dot_general(xc_ref[...], wc_ref[...], dn,
                          preferred_element_type=jnp.float32)
    acc = acc + lax.dot_general(xr_ref[...], wr_ref[...], dn,
                                preferred_element_type=jnp.float32)
    acc = acc + jnp.dot(ea_ref[...], we_ref[...], preferred_element_type=jnp.float32)
    out_ref[...] = jnp.maximum(acc + b1_ref[...], 0.0)


def _edge_mlp(xc, xr, ea, wc, wr, we, b1):
    return pl.pallas_call(
        _mlp_body,
        grid=(E_PAD // MLP_TILE,),
        in_specs=[
            pl.BlockSpec((8, MLP_TILE), lambda i: (0, i)),
            pl.BlockSpec((8, MLP_TILE), lambda i: (0, i)),
            pl.BlockSpec((MLP_TILE, 4), lambda i: (i, 0)),
            pl.BlockSpec((8, H), lambda i: (0, 0)),
            pl.BlockSpec((8, H), lambda i: (0, 0)),
            pl.BlockSpec((4, H), lambda i: (0, 0)),
            pl.BlockSpec((1, H), lambda i: (0, 0)),
        ],
        out_specs=pl.BlockSpec((MLP_TILE, H), lambda i: (i, 0)),
        out_shape=jax.ShapeDtypeStruct((E_PAD, H), jnp.float32),
    )(xc, xr, ea, wc, wr, we, b1)


def _tc_h_body(sp_ref, degp_ref, w2_ref, b2_ref, w10_ref,
               h_ref, z_ref, dinv_ref, oacc_ref):
    sp = sp_ref[...]
    s = sp[0] + sp[1]
    ones = jnp.ones((NW, 1), jnp.float32)
    deg = jnp.dot(degp_ref[...], ones, preferred_element_type=jnp.float32)
    h = jnp.dot(s, w2_ref[...], preferred_element_type=jnp.float32)
    h = h + deg * b2_ref[...]
    safe = jnp.where(deg > 0, deg, 1.0)
    dinv = jnp.where(deg > 0, lax.rsqrt(safe), 0.0)
    h_ref[...] = h
    z_ref[...] = dinv * h
    dinv_ref[...] = dinv
    oacc_ref[...] = jnp.dot(h, w10_ref[...], preferred_element_type=jnp.float32)


def _tc_h(sp, degp, w2, b2, w10):
    return pl.pallas_call(
        _tc_h_body,
        grid=(N // TN,),
        in_specs=[
            pl.BlockSpec((NC, TN, H), lambda i: (0, i, 0)),
            pl.BlockSpec((TN, NW), lambda i: (i, 0)),
            pl.BlockSpec((H, H), lambda i: (0, 0)),
            pl.BlockSpec((1, H), lambda i: (0, 0)),
            pl.BlockSpec((H, H), lambda i: (0, 0)),
        ],
        out_specs=[
            pl.BlockSpec((TN, H), lambda i: (i, 0)),
            pl.BlockSpec((TN, H), lambda i: (i, 0)),
            pl.BlockSpec((TN, 1), lambda i: (i, 0)),
            pl.BlockSpec((TN, H), lambda i: (i, 0)),
        ],
        out_shape=[
            jax.ShapeDtypeStruct((N, H), jnp.float32),
            jax.ShapeDtypeStruct((N, H), jnp.float32),
            jax.ShapeDtypeStruct((N, 1), jnp.float32),
            jax.ShapeDtypeStruct((N, H), jnp.float32),
        ],
    )(sp, degp, w2, b2, w10)


def _tc_tag_body(aggp_ref, dinv_ref, w_ref, oacc_ref, oout_ref, z_ref):
    aggp = aggp_ref[...]
    dinv = dinv_ref[...]
    xk = dinv * (aggp[0] + aggp[1])
    oout_ref[...] = oacc_ref[...] + jnp.dot(xk, w_ref[...],
                                            preferred_element_type=jnp.float32)
    z_ref[...] = dinv * xk


def _tc_tag(aggp, dinv, w, oacc):
    return pl.pallas_call(
        _tc_tag_body,
        grid=(N // TN,),
        in_specs=[
            pl.BlockSpec((NC, TN, H), lambda i: (0, i, 0)),
            pl.BlockSpec((TN, 1), lambda i: (i, 0)),
            pl.BlockSpec((H, H), lambda i: (0, 0)),
            pl.BlockSpec((TN, H), lambda i: (i, 0)),
        ],
        out_specs=[
            pl.BlockSpec((TN, H), lambda i: (i, 0)),
            pl.BlockSpec((TN, H), lambda i: (i, 0)),
        ],
        out_shape=[
            jax.ShapeDtypeStruct((N, H), jnp.float32),
            jax.ShapeDtypeStruct((N, H), jnp.float32),
        ],
    )(aggp, dinv, w, oacc)


def _tc_tag_end_body(aggp_ref, dinv_ref, w_ref, oacc_ref, b_ref, wn_ref,
                     z_ref, oout_ref):
    aggp = aggp_ref[...]
    dinv = dinv_ref[...]
    xk = dinv * (aggp[0] + aggp[1])
    o = oacc_ref[...] + jnp.dot(xk, w_ref[...], preferred_element_type=jnp.float32)
    h = jnp.maximum(o + b_ref[...], 0.0)
    z_ref[...] = dinv * h
    oout_ref[...] = jnp.dot(h, wn_ref[...], preferred_element_type=jnp.float32)


def _tc_tag_end(aggp, dinv, w, oacc, b, wn):
    return pl.pallas_call(
        _tc_tag_end_body,
        grid=(N // TN,),
        in_specs=[
            pl.BlockSpec((NC, TN, H), lambda i: (0, i, 0)),
            pl.BlockSpec((TN, 1), lambda i: (i, 0)),
            pl.BlockSpec((H, H), lambda i: (0, 0)),
            pl.BlockSpec((TN, H), lambda i: (i, 0)),
            pl.BlockSpec((1, H), lambda i: (0, 0)),
            pl.BlockSpec((H, H), lambda i: (0, 0)),
        ],
        out_specs=[
            pl.BlockSpec((TN, H), lambda i: (i, 0)),
            pl.BlockSpec((TN, H), lambda i: (i, 0)),
        ],
        out_shape=[
            jax.ShapeDtypeStruct((N, H), jnp.float32),
            jax.ShapeDtypeStruct((N, H), jnp.float32),
        ],
    )(aggp, dinv, w, oacc, b, wn)


def _tc_tag_final_body(aggp_ref, dinv_ref, w_ref, oacc_ref, b_ref,
                       wo_ref, bo_ref, y_ref):
    aggp = aggp_ref[...]
    dinv = dinv_ref[...]
    xk = dinv * (aggp[0] + aggp[1])
    o = oacc_ref[...] + jnp.dot(xk, w_ref[...], preferred_element_type=jnp.float32)
    h = o + b_ref[...]
    y_ref[...] = jnp.dot(h, wo_ref[...], preferred_element_type=jnp.float32) + bo_ref[...]


def _tc_tag_final(aggp, dinv, w, oacc, b, wo, bo):
    return pl.pallas_call(
        _tc_tag_final_body,
        grid=(N // TN,),
        in_specs=[
            pl.BlockSpec((NC, TN, H), lambda i: (0, i, 0)),
            pl.BlockSpec((TN, 1), lambda i: (i, 0)),
            pl.BlockSpec((H, H), lambda i: (0, 0)),
            pl.BlockSpec((TN, H), lambda i: (i, 0)),
            pl.BlockSpec((1, H), lambda i: (0, 0)),
            pl.BlockSpec((H, 2), lambda i: (0, 0)),
            pl.BlockSpec((1, 2), lambda i: (0, 0)),
        ],
        out_specs=pl.BlockSpec((TN, 2), lambda i: (i, 0)),
        out_shape=jax.ShapeDtypeStruct((N, 2), jnp.float32),
    )(aggp, dinv, w, oacc, b, wo, bo)


def kernel(x, edge_index, edge_attr, ea_W1, ea_b1, ea_W2, ea_b2, tag_W, tag_b, out_W, out_b):
    row = edge_index[0]
    col = edge_index[1]

    npad = E_PAD - E
    pad_ids = jnp.arange(npad, dtype=jnp.int32) % NPAD
    row_p = jnp.concatenate([row, pad_ids])
    col_p = jnp.concatenate([col, N + pad_ids])
    ea_p = jnp.pad(edge_attr, ((0, npad), (0, 0)))

    x8 = jnp.pad(x, ((0, NPAD), (0, 1))).reshape(-1)
    zerosH = jnp.zeros((NA, H), jnp.float32)

    xc_g, xr_g, degp = _sc_gather_x(row_p, col_p, x8)
    degp = degp.reshape(NW, NA).T

    wc = jnp.pad(ea_W1[0:7], ((0, 1), (0, 0)))
    wr = jnp.pad(ea_W1[7:14], ((0, 1), (0, 0)))
    we = ea_W1[14:18]
    h1 = _edge_mlp(xc_g, xr_g, ea_p, wc, wr, we, ea_b1.reshape(1, H))

    sp = _sc_agg_linear(h1, row_p, col_p, zerosH)

    h, z, dinv, oacc = _tc_h(sp, degp, ea_W2, ea_b2.reshape(1, H), tag_W[0, 0])

    aggp = _sc_agg_gather(z, row_p, col_p, zerosH)
    oacc, z = _tc_tag(aggp, dinv, tag_W[0, 1], oacc)
    aggp = _sc_agg_gather(z, row_p, col_p, zerosH)
    oacc, z = _tc_tag(aggp, dinv, tag_W[0, 2], oacc)
    aggp = _sc_agg_gather(z, row_p, col_p, zerosH)
    z, oacc = _tc_tag_end(aggp, dinv, tag_W[0, 3], oacc,
                          tag_b[0].reshape(1, H), tag_W[1, 0])
    aggp = _sc_agg_gather(z, row_p, col_p, zerosH)
    oacc, z = _tc_tag(aggp, dinv, tag_W[1, 1], oacc)
    aggp = _sc_agg_gather(z, row_p, col_p, zerosH)
    oacc, z = _tc_tag(aggp, dinv, tag_W[1, 2], oacc)
    aggp = _sc_agg_gather(z, row_p, col_p, zerosH)
    y = _tc_tag_final(aggp, dinv, tag_W[1, 3], oacc,
                      tag_b[1].reshape(1, H), out_W, out_b.reshape(1, 2))

    return y.reshape(1, -1)


# speedup vs baseline: 15.8349x; 1.2675x over previous
import functools

import jax
import jax.numpy as jnp
from jax import lax
from jax.experimental import pallas as pl
from jax.experimental.pallas import tpu as pltpu
from jax.experimental.pallas import tpu_sc as plsc

N = 10000
E = 640000
H = 128
NPAD = 112
NA = N + NPAD
NC = 2
NS = 16
NW = NC * NS
CW = 128
WPW = 157
EPW = WPW * CW
E_PAD = NW * EPW
NA_PER_S = NA // NS
NBUF = 4

MLP_TILE = 2048
TN = 1000

_mesh = plsc.VectorSubcoreMesh(core_axis_name="c", subcore_axis_name="s")


def _worker_id():
    return lax.axis_index("s") * NC + lax.axis_index("c")


@functools.partial(
    pl.kernel,
    mesh=_mesh,
    out_type=[
        jax.ShapeDtypeStruct((7, E_PAD), jnp.float32),
        jax.ShapeDtypeStruct((7, E_PAD), jnp.float32),
        jax.ShapeDtypeStruct((NW, 1, NA), jnp.float32),
    ],
    scratch_types=[
        pltpu.VMEM((NA * 7,), jnp.float32),
        pltpu.VMEM((WPW, CW), jnp.int32),
        pltpu.VMEM((WPW, CW), jnp.int32),
        pltpu.VMEM((7, CW), jnp.float32),
        pltpu.VMEM((7, CW), jnp.float32),
        pltpu.VMEM((7, CW), jnp.float32),
        pltpu.VMEM((7, CW), jnp.float32),
        pltpu.VMEM((NA,), jnp.float32),
        pltpu.SemaphoreType.DMA,
        pltpu.SemaphoreType.DMA,
        pltpu.SemaphoreType.DMA,
        pltpu.SemaphoreType.DMA,
    ],
    compiler_params=pltpu.CompilerParams(needs_layout_passes=False),
)
def _sc_gather_x(row3_hbm, col3_hbm, x7_hbm,
                 xc_hbm, xr_hbm, deg_hbm,
                 x7v, cidx, ridx, cb0, cb1, rb0, rb1, dacc,
                 cs0, cs1, rs0, rs1):
    wid = _worker_id()
    cbs = (cb0, cb1)
    rbs = (rb0, rb1)
    css = (cs0, cs1)
    rss = (rs0, rs1)

    def zbody(i, _):
        dacc[pl.ds(i * 16, 16)] = jnp.zeros((16,), jnp.float32)
        return _

    lax.fori_loop(0, NA // 16, zbody, None)
    pltpu.sync_copy(x7_hbm, x7v)
    pltpu.sync_copy(col3_hbm.at[wid], cidx)
    pltpu.sync_copy(row3_hbm.at[wid], ridx)
    ones16 = jnp.ones((16,), jnp.float32)

    def compute(w, b):
        for j in range(CW // 16):
            c16 = cidx[w, pl.ds(j * 16, 16)]
            r16 = ridx[w, pl.ds(j * 16, 16)]
            plsc.addupdate_scatter(dacc, [c16], ones16)
            c7 = c16 * 7
            r7 = r16 * 7
            for f in range(7):
                cbs[b][f, pl.ds(j * 16, 16)] = plsc.load_gather(x7v, [c7 + f])
                rbs[b][f, pl.ds(j * 16, 16)] = plsc.load_gather(x7v, [r7 + f])

    def issue_out(w, b):
        off = wid * EPW + w * CW
        pltpu.async_copy(cbs[b], xc_hbm.at[:, pl.ds(off, CW)], css[b])
        pltpu.async_copy(rbs[b], xr_hbm.at[:, pl.ds(off, CW)], rss[b])

    def wait_out(w, b):
        off = wid * EPW + w * CW
        pltpu.make_async_copy(cbs[b], xc_hbm.at[:, pl.ds(off, CW)], css[b]).wait()
        pltpu.make_async_copy(rbs[b], xr_hbm.at[:, pl.ds(off, CW)], rss[b]).wait()

    def body(k, _):
        for b in range(2):
            w = 2 * k + b

            @pl.when(w >= 2)
            def _():
                wait_out(w - 2, b)

            compute(w, b)
            issue_out(w, b)
        return _

    lax.fori_loop(0, (WPW - 1) // 2, body, None)
    wait_out(WPW - 3, 0)
    compute(WPW - 1, 0)
    issue_out(WPW - 1, 0)
    wait_out(WPW - 2, 1)
    wait_out(WPW - 1, 0)
    pltpu.sync_copy(dacc, deg_hbm.at[wid, 0])


def _make_sc_aggregate(gather_src: bool):
    scratch = [pltpu.VMEM((CW,), jnp.int32)] * 8
    scratch += [pltpu.VMEM((CW, H), jnp.float32)] * 2
    scratch += [pltpu.VMEM_SHARED((NA, H), jnp.float32)]
    scratch += [pltpu.SemaphoreType.DMA] * 12

    @functools.partial(
        pl.kernel,
        mesh=_mesh,
        out_type=jax.ShapeDtypeStruct((NC, NA, H), jnp.float32),
        scratch_types=scratch,
    )
    def agg(src_hbm, row_hbm, col_hbm, zeros_hbm, out_hbm,
            c0, c1, c2, c3, r0, r1, r2, r3, b0, b1, acc,
            ic0, ic1, ic2, ic3, ir0, ir1, ir2, ir3, g0, g1, s0, s1):
        cid = lax.axis_index("c")
        sid = lax.axis_index("s")
        wid = _worker_id()
        cidx = (c0, c1, c2, c3)
        ridx = (r0, r1, r2, r3)
        bufs = (b0, b1)
        icsem = (ic0, ic1, ic2, ic3)
        irsem = (ir0, ir1, ir2, ir3)
        gsem = (g0, g1)
        ssem = (s0, s1)

        pltpu.sync_copy(zeros_hbm.at[pl.ds(sid * NA_PER_S, NA_PER_S)],
                        acc.at[pl.ds(sid * NA_PER_S, NA_PER_S)])
        plsc.subcore_barrier()

        def issue_idx(w, i):
            off = wid * EPW + w * CW
            pltpu.async_copy(col_hbm.at[pl.ds(off, CW)], cidx[i], icsem[i])
            if gather_src:
                pltpu.async_copy(row_hbm.at[pl.ds(off, CW)], ridx[i], irsem[i])

        def wait_idx(w, i):
            off = wid * EPW + w * CW
            pltpu.make_async_copy(col_hbm.at[pl.ds(off, CW)], cidx[i], icsem[i]).wait()
            if gather_src:
                pltpu.make_async_copy(row_hbm.at[pl.ds(off, CW)], ridx[i], irsem[i]).wait()

        def gsrc(w, i):
            if gather_src:
                return src_hbm.at[ridx[i]]
            return src_hbm.at[pl.ds(wid * EPW + w * CW, CW)]

        def issue_gather(w, i, b):
            pltpu.async_copy(gsrc(w, i), bufs[b], gsem[b])

        def wait_gather(w, i, b):
            pltpu.make_async_copy(gsrc(w, i), bufs[b], gsem[b]).wait()

        def issue_scatter(w, i, b):
            pltpu.async_copy(bufs[b], acc.at[cidx[i]], ssem[b], add=True)

        def wait_scatter(w, i, b):
            pltpu.make_async_copy(bufs[b], acc.at[cidx[i]], ssem[b]).wait()

        issue_idx(0, 0)
        issue_idx(1, 1)

        def slot(w, wi, b):
            @pl.when(w >= 2)
            def _():
                wait_scatter(w - 2, (wi - 2) % 4, b)

            wait_idx(w, wi)
            issue_gather(w, wi, b)

            @pl.when(w + 2 < WPW)
            def _():
                issue_idx(w + 2, (wi + 2) % 4)

            @pl.when(w >= 1)
            def _():
                wait_gather(w - 1, (wi - 1) % 4, 1 - b)
                issue_scatter(w - 1, (wi - 1) % 4, 1 - b)

        def body(k, _):
            w0 = 4 * k
            for j in range(4):
                slot(w0 + j, j, j % 2)
            return _

        lax.fori_loop(0, WPW // 4, body, None)
        w = WPW - 1
        wait_scatter(w - 2, 2, 0)
        wait_idx(w, 0)
        issue_gather(w, 0, 0)
        wait_gather(w - 1, 3, 1)
        issue_scatter(w - 1, 3, 1)
        wait_gather(w, 0, 0)
        issue_scatter(w, 0, 0)
        wait_scatter(w - 1, 3, 1)
        wait_scatter(w, 0, 0)

        plsc.subcore_barrier()
        pltpu.sync_copy(acc.at[pl.ds(sid * NA_PER_S, NA_PER_S)],
                        out_hbm.at[cid, pl.ds(sid * NA_PER_S, NA_PER_S)])

    return agg


_sc_agg_linear = _make_sc_aggregate(gather_src=False)
_sc_agg_gather = _make_sc_aggregate(gather_src=True)


def _mlp_body(xc_ref, xr_ref, ea_ref, wc_ref, wr_ref, we_ref, b1_ref, out_ref):
    dn = (((0,), (0,)), ((), ()))
    acc = lax.dot_general(xc_ref[...], wc_ref[...], dn,
                          preferred_element_type=jnp.float32)
    acc = acc + lax.dot_general(xr_ref[...], wr_ref[...], dn,
                                preferred_element_type=jnp.float32)
    acc = acc + jnp.dot(ea_ref[...], we_ref[...], preferred_element_type=jnp.float32)
    out_ref[...] = jnp.maximum(acc + b1_ref[...], 0.0)


def _edge_mlp(xc, xr, ea, wc, wr, we, b1):
    return pl.pallas_call(
        _mlp_body,
        grid=(E_PAD // MLP_TILE,),
        in_specs=[
            pl.BlockSpec((7, MLP_TILE), lambda i: (0, i)),
            pl.BlockSpec((7, MLP_TILE), lambda i: (0, i)),
            pl.BlockSpec((MLP_TILE, 4), lambda i: (i, 0)),
            pl.BlockSpec((7, H), lambda i: (0, 0)),
            pl.BlockSpec((7, H), lambda i: (0, 0)),
            pl.BlockSpec((4, H), lambda i: (0, 0)),
            pl.BlockSpec((1, H), lambda i: (0, 0)),
        ],
        out_specs=pl.BlockSpec((MLP_TILE, H), lambda i: (i, 0)),
        out_shape=jax.ShapeDtypeStruct((E_PAD, H), jnp.float32),
    )(xc, xr, ea, wc, wr, we, b1)


def _tc_h_body(sp_ref, deg_ref, w2_ref, b2_ref, w10_ref,
               h_ref, z_ref, dinv_ref, oacc_ref):
    sp = sp_ref[...]
    s = sp[0] + sp[1]
    deg = deg_ref[...]
    h = jnp.dot(s, w2_ref[...], preferred_element_type=jnp.float32)
    h = h + deg * b2_ref[...]
    safe = jnp.where(deg > 0, deg, 1.0)
    dinv = jnp.where(deg > 0, lax.rsqrt(safe), 0.0)
    h_ref[...] = h
    z_ref[...] = dinv * h
    dinv_ref[...] = dinv
    oacc_ref[...] = jnp.dot(h, w10_ref[...], preferred_element_type=jnp.float32)


def _tc_h(sp, deg, w2, b2, w10):
    return pl.pallas_call(
        _tc_h_body,
        grid=(N // TN,),
        in_specs=[
            pl.BlockSpec((NC, TN, H), lambda i: (0, i, 0)),
            pl.BlockSpec((TN, 1), lambda i: (i, 0)),
            pl.BlockSpec((H, H), lambda i: (0, 0)),
            pl.BlockSpec((1, H), lambda i: (0, 0)),
            pl.BlockSpec((H, H), lambda i: (0, 0)),
        ],
        out_specs=[
            pl.BlockSpec((TN, H), lambda i: (i, 0)),
            pl.BlockSpec((TN, H), lambda i: (i, 0)),
            pl.BlockSpec((TN, 1), lambda i: (i, 0)),
            pl.BlockSpec((TN, H), lambda i: (i, 0)),
        ],
        out_shape=[
            jax.ShapeDtypeStruct((N, H), jnp.float32),
            jax.ShapeDtypeStruct((N, H), jnp.float32),
            jax.ShapeDtypeStruct((N, 1), jnp.float32),
            jax.ShapeDtypeStruct((N, H), jnp.float32),
        ],
    )(sp, deg, w2, b2, w10)


def _tc_tag_body(aggp_ref, dinv_ref, w_ref, oacc_ref, oout_ref, z_ref):
    aggp = aggp_ref[...]
    dinv = dinv_ref[...]
    xk = dinv * (aggp[0] + aggp[1])
    oout_ref[...] = oacc_ref[...] + jnp.dot(xk, w_ref[...],
                                            preferred_element_type=jnp.float32)
    z_ref[...] = dinv * xk


def _tc_tag(aggp, dinv, w, oacc):
    return pl.pallas_call(
        _tc_tag_body,
        grid=(N // TN,),
        in_specs=[
            pl.BlockSpec((NC, TN, H), lambda i: (0, i, 0)),
            pl.BlockSpec((TN, 1), lambda i: (i, 0)),
            pl.BlockSpec((H, H), lambda i: (0, 0)),
            pl.BlockSpec((TN, H), lambda i: (i, 0)),
        ],
        out_specs=[
            pl.BlockSpec((TN, H), lambda i: (i, 0)),
            pl.BlockSpec((TN, H), lambda i: (i, 0)),
        ],
        out_shape=[
            jax.ShapeDtypeStruct((N, H), jnp.float32),
            jax.ShapeDtypeStruct((N, H), jnp.float32),
        ],
    )(aggp, dinv, w, oacc)


def _tc_tag_end_body(aggp_ref, dinv_ref, w_ref, oacc_ref, b_ref, wn_ref,
                     z_ref, oout_ref):
    aggp = aggp_ref[...]
    dinv = dinv_ref[...]
    xk = dinv * (aggp[0] + aggp[1])
    o = oacc_ref[...] + jnp.dot(xk, w_ref[...], preferred_element_type=jnp.float32)
    h = jnp.maximum(o + b_ref[...], 0.0)
    z_ref[...] = dinv * h
    oout_ref[...] = jnp.dot(h, wn_ref[...], preferred_element_type=jnp.float32)


def _tc_tag_end(aggp, dinv, w, oacc, b, wn):
    return pl.pallas_call(
        _tc_tag_end_body,
        grid=(N // TN,),
        in_specs=[
            pl.BlockSpec((NC, TN, H), lambda i: (0, i, 0)),
            pl.BlockSpec((TN, 1), lambda i: (i, 0)),
            pl.BlockSpec((H, H), lambda i: (0, 0)),
            pl.BlockSpec((TN, H), lambda i: (i, 0)),
            pl.BlockSpec((1, H), lambda i: (0, 0)),
            pl.BlockSpec((H, H), lambda i: (0, 0)),
        ],
        out_specs=[
            pl.BlockSpec((TN, H), lambda i: (i, 0)),
            pl.BlockSpec((TN, H), lambda i: (i, 0)),
        ],
        out_shape=[
            jax.ShapeDtypeStruct((N, H), jnp.float32),
            jax.ShapeDtypeStruct((N, H), jnp.float32),
        ],
    )(aggp, dinv, w, oacc, b, wn)


def _tc_tag_final_body(aggp_ref, dinv_ref, w_ref, oacc_ref, b_ref,
                       wo_ref, bo_ref, y_ref):
    aggp = aggp_ref[...]
    dinv = dinv_ref[...]
    xk = dinv * (aggp[0] + aggp[1])
    o = oacc_ref[...] + jnp.dot(xk, w_ref[...], preferred_element_type=jnp.float32)
    h = o + b_ref[...]
    y_ref[...] = jnp.dot(h, wo_ref[...], preferred_element_type=jnp.float32) + bo_ref[...]


def _tc_tag_final(aggp, dinv, w, oacc, b, wo, bo):
    return pl.pallas_call(
        _tc_tag_final_body,
        grid=(N // TN,),
        in_specs=[
            pl.BlockSpec((NC, TN, H), lambda i: (0, i, 0)),
            pl.BlockSpec((TN, 1), lambda i: (i, 0)),
            pl.BlockSpec((H, H), lambda i: (0, 0)),
            pl.BlockSpec((TN, H), lambda i: (i, 0)),
            pl.BlockSpec((1, H), lambda i: (0, 0)),
            pl.BlockSpec((H, 2), lambda i: (0, 0)),
            pl.BlockSpec((1, 2), lambda i: (0, 0)),
        ],
        out_specs=pl.BlockSpec((TN, 2), lambda i: (i, 0)),
        out_shape=jax.ShapeDtypeStruct((N, 2), jnp.float32),
    )(aggp, dinv, w, oacc, b, wo, bo)


def kernel(x, edge_index, edge_attr, ea_W1, ea_b1, ea_W2, ea_b2, tag_W, tag_b, out_W, out_b):
    row = edge_index[0]
    col = edge_index[1]

    npad = E_PAD - E
    pad_ids = jnp.arange(npad, dtype=jnp.int32) % NPAD
    row_p = jnp.concatenate([row, pad_ids])
    col_p = jnp.concatenate([col, N + pad_ids])
    row3 = row_p.reshape(NW, WPW, CW)
    col3 = col_p.reshape(NW, WPW, CW)
    ea_p = jnp.pad(edge_attr, ((0, npad), (0, 0)))

    x7 = jnp.pad(x, ((0, NPAD), (0, 0))).reshape(-1)
    zerosH = jnp.zeros((NA, H), jnp.float32)

    xc_g, xr_g, degp = _sc_gather_x(row3, col3, x7)
    deg = degp.reshape(NW, NA).sum(axis=0)[:N].reshape(N, 1)

    wc = ea_W1[0:7]
    wr = ea_W1[7:14]
    we = ea_W1[14:18]
    h1 = _edge_mlp(xc_g, xr_g, ea_p, wc, wr, we, ea_b1.reshape(1, H))

    sp = _sc_agg_linear(h1, row_p, col_p, zerosH)

    h, z, dinv, oacc = _tc_h(sp, deg, ea_W2, ea_b2.reshape(1, H), tag_W[0, 0])

    aggp = _sc_agg_gather(z, row_p, col_p, zerosH)
    oacc, z = _tc_tag(aggp, dinv, tag_W[0, 1], oacc)
    aggp = _sc_agg_gather(z, row_p, col_p, zerosH)
    oacc, z = _tc_tag(aggp, dinv, tag_W[0, 2], oacc)
    aggp = _sc_agg_gather(z, row_p, col_p, zerosH)
    z, oacc = _tc_tag_end(aggp, dinv, tag_W[0, 3], oacc,
                          tag_b[0].reshape(1, H), tag_W[1, 0])
    aggp = _sc_agg_gather(z, row_p, col_p, zerosH)
    oacc, z = _tc_tag(aggp, dinv, tag_W[1, 1], oacc)
    aggp = _sc_agg_gather(z, row_p, col_p, zerosH)
    oacc, z = _tc_tag(aggp, dinv, tag_W[1, 2], oacc)
    aggp = _sc_agg_gather(z, row_p, col_p, zerosH)
    y = _tc_tag_final(aggp, dinv, tag_W[1, 3], oacc,
                      tag_b[1].reshape(1, H), out_W, out_b.reshape(1, 2))

    return y.reshape(1, -1)


# speedup vs baseline: 15.8453x; 1.0007x over previous
import functools

import jax
import jax.numpy as jnp
from jax import lax
from jax.experimental import pallas as pl
from jax.experimental.pallas import tpu as pltpu
from jax.experimental.pallas import tpu_sc as plsc

N = 10000
E = 640000
H = 128
NPAD = 112
NA = N + NPAD
NC = 2
NS = 16
NW = NC * NS
CW = 128
WPW = 157
EPW = WPW * CW
E_PAD = NW * EPW
NA_PER_S = NA // NS
NBUF = 4

MLP_TILE = 2048
TN = 1000

_mesh = plsc.VectorSubcoreMesh(core_axis_name="c", subcore_axis_name="s")


def _worker_id():
    return lax.axis_index("s") * NC + lax.axis_index("c")


@functools.partial(
    pl.kernel,
    mesh=_mesh,
    out_type=[
        jax.ShapeDtypeStruct((7, E_PAD), jnp.float32),
        jax.ShapeDtypeStruct((7, E_PAD), jnp.float32),
        jax.ShapeDtypeStruct((NW, 1, NA), jnp.float32),
    ],
    scratch_types=[
        pltpu.VMEM((NA * 7,), jnp.float32),
        pltpu.VMEM((WPW, CW), jnp.int32),
        pltpu.VMEM((WPW, CW), jnp.int32),
        pltpu.VMEM((7, CW), jnp.float32),
        pltpu.VMEM((7, CW), jnp.float32),
        pltpu.VMEM((7, CW), jnp.float32),
        pltpu.VMEM((7, CW), jnp.float32),
        pltpu.VMEM((NA,), jnp.float32),
        pltpu.SemaphoreType.DMA,
        pltpu.SemaphoreType.DMA,
        pltpu.SemaphoreType.DMA,
        pltpu.SemaphoreType.DMA,
    ],
    compiler_params=pltpu.CompilerParams(needs_layout_passes=False),
)
def _sc_gather_x(row3_hbm, col3_hbm, x7_hbm,
                 xc_hbm, xr_hbm, deg_hbm,
                 x7v, cidx, ridx, cb0, cb1, rb0, rb1, dacc,
                 cs0, cs1, rs0, rs1):
    wid = _worker_id()
    cbs = (cb0, cb1)
    rbs = (rb0, rb1)
    css = (cs0, cs1)
    rss = (rs0, rs1)

    def zbody(i, _):
        dacc[pl.ds(i * 16, 16)] = jnp.zeros((16,), jnp.float32)
        return _

    lax.fori_loop(0, NA // 16, zbody, None)
    pltpu.sync_copy(x7_hbm, x7v)
    pltpu.sync_copy(col3_hbm.at[wid], cidx)
    pltpu.sync_copy(row3_hbm.at[wid], ridx)
    ones16 = jnp.ones((16,), jnp.float32)

    def compute(w, b):
        for j in range(CW // 16):
            c16 = cidx[w, pl.ds(j * 16, 16)]
            r16 = ridx[w, pl.ds(j * 16, 16)]
            plsc.addupdate_scatter(dacc, [c16], ones16)
            c7 = c16 * 7
            r7 = r16 * 7
            for f in range(7):
                cbs[b][f, pl.ds(j * 16, 16)] = plsc.load_gather(x7v, [c7 + f])
                rbs[b][f, pl.ds(j * 16, 16)] = plsc.load_gather(x7v, [r7 + f])

    def issue_out(w, b):
        off = wid * EPW + w * CW
        pltpu.async_copy(cbs[b], xc_hbm.at[:, pl.ds(off, CW)], css[b])
        pltpu.async_copy(rbs[b], xr_hbm.at[:, pl.ds(off, CW)], rss[b])

    def wait_out(w, b):
        off = wid * EPW + w * CW
        pltpu.make_async_copy(cbs[b], xc_hbm.at[:, pl.ds(off, CW)], css[b]).wait()
        pltpu.make_async_copy(rbs[b], xr_hbm.at[:, pl.ds(off, CW)], rss[b]).wait()

    def body(k, _):
        for b in range(2):
            w = 2 * k + b

            @pl.when(w >= 2)
            def _():
                wait_out(w - 2, b)

            compute(w, b)
            issue_out(w, b)
        return _

    lax.fori_loop(0, (WPW - 1) // 2, body, None)
    wait_out(WPW - 3, 0)
    compute(WPW - 1, 0)
    issue_out(WPW - 1, 0)
    wait_out(WPW - 2, 1)
    wait_out(WPW - 1, 0)
    pltpu.sync_copy(dacc, deg_hbm.at[wid, 0])


def _make_sc_aggregate(gather_src: bool):
    scratch = [pltpu.VMEM((CW,), jnp.int32)] * 8
    scratch += [pltpu.VMEM((CW, H), jnp.float32)] * 2
    scratch += [pltpu.VMEM_SHARED((NA, H), jnp.float32)]
    scratch += [pltpu.SemaphoreType.DMA] * 12

    @functools.partial(
        pl.kernel,
        mesh=_mesh,
        out_type=jax.ShapeDtypeStruct((NC, NA, H), jnp.float32),
        scratch_types=scratch,
    )
    def agg(src_hbm, row_hbm, col_hbm, zeros_hbm, out_hbm,
            c0, c1, c2, c3, r0, r1, r2, r3, b0, b1, acc,
            ic0, ic1, ic2, ic3, ir0, ir1, ir2, ir3, g0, g1, s0, s1):
        cid = lax.axis_index("c")
        sid = lax.axis_index("s")
        wid = _worker_id()
        cidx = (c0, c1, c2, c3)
        ridx = (r0, r1, r2, r3)
        bufs = (b0, b1)
        icsem = (ic0, ic1, ic2, ic3)
        irsem = (ir0, ir1, ir2, ir3)
        gsem = (g0, g1)
        ssem = (s0, s1)

        pltpu.sync_copy(zeros_hbm.at[pl.ds(sid * NA_PER_S, NA_PER_S)],
                        acc.at[pl.ds(sid * NA_PER_S, NA_PER_S)])
        plsc.subcore_barrier()

        def issue_idx(w, i):
            off = wid * EPW + w * CW
            pltpu.async_copy(col_hbm.at[pl.ds(off, CW)], cidx[i], icsem[i])
            if gather_src:
                pltpu.async_copy(row_hbm.at[pl.ds(off, CW)], ridx[i], irsem[i])

        def wait_idx(w, i):
            off = wid * EPW + w * CW
            pltpu.make_async_copy(col_hbm.at[pl.ds(off, CW)], cidx[i], icsem[i]).wait()
            if gather_src:
                pltpu.make_async_copy(row_hbm.at[pl.ds(off, CW)], ridx[i], irsem[i]).wait()

        def gsrc(w, i):
            if gather_src:
                return src_hbm.at[ridx[i]]
            return src_hbm.at[pl.ds(wid * EPW + w * CW, CW)]

        def issue_gather(w, i, b):
            pltpu.async_copy(gsrc(w, i), bufs[b], gsem[b])

        def wait_gather(w, i, b):
            pltpu.make_async_copy(gsrc(w, i), bufs[b], gsem[b]).wait()

        def issue_scatter(w, i, b):
            pltpu.async_copy(bufs[b], acc.at[cidx[i]], ssem[b], add=True)

        def wait_scatter(w, i, b):
            pltpu.make_async_copy(bufs[b], acc.at[cidx[i]], ssem[b]).wait()

        issue_idx(0, 0)
        issue_idx(1, 1)

        def slot(w, wi, b):
            @pl.when(w >= 2)
            def _():
                wait_scatter(w - 2, (wi - 2) % 4, b)

            wait_idx(w, wi)
            issue_gather(w, wi, b)

            @pl.when(w + 2 < WPW)
            def _():
                issue_idx(w + 2, (wi + 2) % 4)

            @pl.when(w >= 1)
            def _():
                wait_gather(w - 1, (wi - 1) % 4, 1 - b)
                issue_scatter(w - 1, (wi - 1) % 4, 1 - b)

        def body(k, _):
            w0 = 4 * k
            for j in range(4):
                slot(w0 + j, j, j % 2)
            return _

        lax.fori_loop(0, WPW // 4, body, None)
        w = WPW - 1
        wait_scatter(w - 2, 2, 0)
        wait_idx(w, 0)
        issue_gather(w, 0, 0)
        wait_gather(w - 1, 3, 1)
        issue_scatter(w - 1, 3, 1)
        wait_gather(w, 0, 0)
        issue_scatter(w, 0, 0)
        wait_scatter(w - 1, 3, 1)
        wait_scatter(w, 0, 0)

        plsc.subcore_barrier()
        pltpu.sync_copy(acc.at[pl.ds(sid * NA_PER_S, NA_PER_S)],
                        out_hbm.at[cid, pl.ds(sid * NA_PER_S, NA_PER_S)])

    return agg


_sc_agg_linear = _make_sc_aggregate(gather_src=False)
_sc_agg_gather = _make_sc_aggregate(gather_src=True)


def _mlp_body(xc_ref, xr_ref, ea_ref, wc_ref, wr_ref, we_ref, b1_ref, out_ref):
    dn = (((0,), (0,)), ((), ()))
    acc = lax.dot_general(xc_ref[...], wc_ref[...], dn,
                          preferred_element_type=jnp.float32)
    acc = acc + lax.dot_general(xr_ref[...], wr_ref[...], dn,
                                preferred_element_type=jnp.float32)
    acc = acc + jnp.dot(ea_ref[...], we_ref[...], preferred_element_type=jnp.float32)
    out_ref[...] = jnp.maximum(acc + b1_ref[...], 0.0).reshape(MLP_TILE // 8, 8, H)


def _edge_mlp(xc, xr, ea, wc, wr, we, b1):
    return pl.pallas_call(
        _mlp_body,
        grid=(E_PAD // MLP_TILE,),
        in_specs=[
            pl.BlockSpec((7, MLP_TILE), lambda i: (0, i)),
            pl.BlockSpec((7, MLP_TILE), lambda i: (0, i)),
            pl.BlockSpec((MLP_TILE, 4), lambda i: (i, 0)),
            pl.BlockSpec((7, H), lambda i: (0, 0)),
            pl.BlockSpec((7, H), lambda i: (0, 0)),
            pl.BlockSpec((4, H), lambda i: (0, 0)),
            pl.BlockSpec((1, H), lambda i: (0, 0)),
        ],
        out_specs=pl.BlockSpec((MLP_TILE // 8, 8, H), lambda i: (i, 0, 0)),
        out_shape=jax.ShapeDtypeStruct((E_PAD // 8, 8, H), jnp.float32),
    )(xc, xr, ea, wc, wr, we, b1)


def _tc_h_body(sp_ref, deg_ref, w2_ref, b2_ref, w10_ref,
               h_ref, z_ref, dinv_ref, oacc_ref):
    sp = sp_ref[...]
    s = sp[0] + sp[1]
    deg = deg_ref[...]
    h = jnp.dot(s, w2_ref[...], preferred_element_type=jnp.float32)
    h = h + deg * b2_ref[...]
    safe = jnp.where(deg > 0, deg, 1.0)
    dinv = jnp.where(deg > 0, lax.rsqrt(safe), 0.0)
    h_ref[...] = h
    z_ref[...] = dinv * h
    dinv_ref[...] = dinv
    oacc_ref[...] = jnp.dot(h, w10_ref[...], preferred_element_type=jnp.float32)


def _tc_h(sp, deg, w2, b2, w10):
    return pl.pallas_call(
        _tc_h_body,
        grid=(N // TN,),
        in_specs=[
            pl.BlockSpec((NC, TN, H), lambda i: (0, i, 0)),
            pl.BlockSpec((TN, 1), lambda i: (i, 0)),
            pl.BlockSpec((H, H), lambda i: (0, 0)),
            pl.BlockSpec((1, H), lambda i: (0, 0)),
            pl.BlockSpec((H, H), lambda i: (0, 0)),
        ],
        out_specs=[
            pl.BlockSpec((TN, H), lambda i: (i, 0)),
            pl.BlockSpec((TN, H), lambda i: (i, 0)),
            pl.BlockSpec((TN, 1), lambda i: (i, 0)),
            pl.BlockSpec((TN, H), lambda i: (i, 0)),
        ],
        out_shape=[
            jax.ShapeDtypeStruct((N, H), jnp.float32),
            jax.ShapeDtypeStruct((N, H), jnp.float32),
            jax.ShapeDtypeStruct((N, 1), jnp.float32),
            jax.ShapeDtypeStruct((N, H), jnp.float32),
        ],
    )(sp, deg, w2, b2, w10)


def _tc_tag_body(aggp_ref, dinv_ref, w_ref, oacc_ref, oout_ref, z_ref):
    aggp = aggp_ref[...]
    dinv = dinv_ref[...]
    xk = dinv * (aggp[0] + aggp[1])
    oout_ref[...] = oacc_ref[...] + jnp.dot(xk, w_ref[...],
                                            preferred_element_type=jnp.float32)
    z_ref[...] = dinv * xk


def _tc_tag(aggp, dinv, w, oacc):
    return pl.pallas_call(
        _tc_tag_body,
        grid=(N // TN,),
        in_specs=[
            pl.BlockSpec((NC, TN, H), lambda i: (0, i, 0)),
            pl.BlockSpec((TN, 1), lambda i: (i, 0)),
            pl.BlockSpec((H, H), lambda i: (0, 0)),
            pl.BlockSpec((TN, H), lambda i: (i, 0)),
        ],
        out_specs=[
            pl.BlockSpec((TN, H), lambda i: (i, 0)),
            pl.BlockSpec((TN, H), lambda i: (i, 0)),
        ],
        out_shape=[
            jax.ShapeDtypeStruct((N, H), jnp.float32),
            jax.ShapeDtypeStruct((N, H), jnp.float32),
        ],
    )(aggp, dinv, w, oacc)


def _tc_tag_end_body(aggp_ref, dinv_ref, w_ref, oacc_ref, b_ref, wn_ref,
                     z_ref, oout_ref):
    aggp = aggp_ref[...]
    dinv = dinv_ref[...]
    xk = dinv * (aggp[0] + aggp[1])
    o = oacc_ref[...] + jnp.dot(xk, w_ref[...], preferred_element_type=jnp.float32)
    h = jnp.maximum(o + b_ref[...], 0.0)
    z_ref[...] = dinv * h
    oout_ref[...] = jnp.dot(h, wn_ref[...], preferred_element_type=jnp.float32)


def _tc_tag_end(aggp, dinv, w, oacc, b, wn):
    return pl.pallas_call(
        _tc_tag_end_body,
        grid=(N // TN,),
        in_specs=[
            pl.BlockSpec((NC, TN, H), lambda i: (0, i, 0)),
            pl.BlockSpec((TN, 1), lambda i: (i, 0)),
            pl.BlockSpec((H, H), lambda i: (0, 0)),
            pl.BlockSpec((TN, H), lambda i: (i, 0)),
            pl.BlockSpec((1, H), lambda i: (0, 0)),
            pl.BlockSpec((H, H), lambda i: (0, 0)),
        ],
        out_specs=[
            pl.BlockSpec((TN, H), lambda i: (i, 0)),
            pl.BlockSpec((TN, H), lambda i: (i, 0)),
        ],
        out_shape=[
            jax.ShapeDtypeStruct((N, H), jnp.float32),
            jax.ShapeDtypeStruct((N, H), jnp.float32),
        ],
    )(aggp, dinv, w, oacc, b, wn)


def _tc_tag_final_body(aggp_ref, dinv_ref, w_ref, oacc_ref, b_ref,
                       wo_ref, bo_ref, y_ref):
    aggp = aggp_ref[...]
    dinv = dinv_ref[...]
    xk = dinv * (aggp[0] + aggp[1])
    o = oacc_ref[...] + jnp.dot(xk, w_ref[...], preferred_element_type=jnp.float32)
    h = o + b_ref[...]
    y_ref[...] = jnp.dot(h, wo_ref[...], preferred_element_type=jnp.float32) + bo_ref[...]


def _tc_tag_final(aggp, dinv, w, oacc, b, wo, bo):
    return pl.pallas_call(
        _tc_tag_final_body,
        grid=(N // TN,),
        in_specs=[
            pl.BlockSpec((NC, TN, H), lambda i: (0, i, 0)),
            pl.BlockSpec((TN, 1), lambda i: (i, 0)),
            pl.BlockSpec((H, H), lambda i: (0, 0)),
            pl.BlockSpec((TN, H), lambda i: (i, 0)),
            pl.BlockSpec((1, H), lambda i: (0, 0)),
            pl.BlockSpec((H, 2), lambda i: (0, 0)),
            pl.BlockSpec((1, 2), lambda i: (0, 0)),
        ],
        out_specs=pl.BlockSpec((TN, 2), lambda i: (i, 0)),
        out_shape=jax.ShapeDtypeStruct((N, 2), jnp.float32),
    )(aggp, dinv, w, oacc, b, wo, bo)


def kernel(x, edge_index, edge_attr, ea_W1, ea_b1, ea_W2, ea_b2, tag_W, tag_b, out_W, out_b):
    row = edge_index[0]
    col = edge_index[1]

    npad = E_PAD - E
    pad_ids = jnp.arange(npad, dtype=jnp.int32) % NPAD
    row_p = jnp.concatenate([row, pad_ids])
    col_p = jnp.concatenate([col, N + pad_ids])
    row3 = row_p.reshape(NW, WPW, CW)
    col3 = col_p.reshape(NW, WPW, CW)
    ea_p = jnp.pad(edge_attr, ((0, npad), (0, 0)))

    x7 = jnp.pad(x, ((0, NPAD), (0, 0))).reshape(-1)
    zerosH = jnp.zeros((NA, H), jnp.float32)

    xc_g, xr_g, degp = _sc_gather_x(row3, col3, x7)
    deg = degp.reshape(NW, NA).sum(axis=0)[:N].reshape(N, 1)

    wc = ea_W1[0:7]
    wr = ea_W1[7:14]
    we = ea_W1[14:18]
    h1 = _edge_mlp(xc_g, xr_g, ea_p, wc, wr, we, ea_b1.reshape(1, H))
    h1 = h1.reshape(E_PAD, H)

    sp = _sc_agg_linear(h1, row_p, col_p, zerosH)

    h, z, dinv, oacc = _tc_h(sp, deg, ea_W2, ea_b2.reshape(1, H), tag_W[0, 0])

    aggp = _sc_agg_gather(z, row_p, col_p, zerosH)
    oacc, z = _tc_tag(aggp, dinv, tag_W[0, 1], oacc)
    aggp = _sc_agg_gather(z, row_p, col_p, zerosH)
    oacc, z = _tc_tag(aggp, dinv, tag_W[0, 2], oacc)
    aggp = _sc_agg_gather(z, row_p, col_p, zerosH)
    z, oacc = _tc_tag_end(aggp, dinv, tag_W[0, 3], oacc,
                          tag_b[0].reshape(1, H), tag_W[1, 0])
    aggp = _sc_agg_gather(z, row_p, col_p, zerosH)
    oacc, z = _tc_tag(aggp, dinv, tag_W[1, 1], oacc)
    aggp = _sc_agg_gather(z, row_p, col_p, zerosH)
    oacc, z = _tc_tag(aggp, dinv, tag_W[1, 2], oacc)
    aggp = _sc_agg_gather(z, row_p, col_p, zerosH)
    y = _tc_tag_final(aggp, dinv, tag_W[1, 3], oacc,
                      tag_b[1].reshape(1, H), out_W, out_b.reshape(1, 2))

    return y.reshape(1, -1)
